# baseline (speedup 1.0000x reference)
import functools

import jax
import jax.numpy as jnp
from jax import lax
from jax.experimental import pallas as pl
from jax.experimental.pallas import tpu as pltpu
from jax.experimental.pallas import tpu_sc as plsc

_N = 10000
_DEG = 16
_D = 128
_L = 3

_NC, _NS = 2, 16
_NW = _NC * _NS
_CHUNK = 128
_NPAD = 10240
_CPW = (_NPAD * _DEG) // _NW // _CHUNK


@functools.cache
def _make_sc_gather():
    mesh = plsc.VectorSubcoreMesh(
        core_axis_name="c", subcore_axis_name="s",
        num_cores=_NC, num_subcores=_NS)

    @functools.partial(
        pl.kernel,
        mesh=mesh,
        out_type=jax.ShapeDtypeStruct((_NW * _CPW, _CHUNK, _D), jnp.float32),
        scratch_types=[
            pltpu.VMEM((_CPW, _CHUNK), jnp.int32),
            pltpu.VMEM((4, _CHUNK, _D), jnp.float32),
            pltpu.SemaphoreType.DMA,
            pltpu.SemaphoreType.DMA,
            pltpu.SemaphoreType.DMA,
            pltpu.SemaphoreType.DMA,
        ],
    )
    def _sc_gather(table_hbm, idx_hbm, out_hbm, idx_all, rows, s0, s1, s2, s3):
        wid = lax.axis_index("s") * _NC + lax.axis_index("c")
        base = wid * _CPW
        sem = (s0, s1, s2, s3)
        pltpu.sync_copy(idx_hbm.at[wid], idx_all)
        gd = [None] * 4
        wd = [None] * 4
        for j in range(2):
            gd[j] = pltpu.async_copy(table_hbm.at[idx_all.at[j]], rows.at[j],
                                     sem[j])
        for j in range(_CPW):
            b = j & 3
            gd[b].wait()
            nj = j + 2
            if nj < _CPW:
                nb = nj & 3
                if wd[nb] is not None:
                    wd[nb].wait()
                    wd[nb] = None
                gd[nb] = pltpu.async_copy(table_hbm.at[idx_all.at[nj]],
                                          rows.at[nb], sem[nb])
            wd[b] = pltpu.async_copy(rows.at[b], out_hbm.at[base + j], sem[b])
        for b in range(4):
            if wd[b] is not None:
                wd[b].wait()

    return _sc_gather


_BN = 1000


def _lstm_body(msg_ref, h_ref, wcat_ref, bias_ref, wcomb_ref, bout_ref,
               out_ref):
    bf = jnp.bfloat16
    c = None
    h2 = jnp.zeros((_BN, _D), bf)
    for t in range(_DEG):
        cat = jnp.concatenate([msg_ref[t].astype(bf), h2], axis=1)
        gates = jnp.dot(cat, wcat_ref[...],
                        preferred_element_type=jnp.float32) + bias_ref[...]
        ti = jnp.tanh(gates[:, :_D])
        g = jnp.tanh(gates[:, 2 * _D:3 * _D])
        if c is None:
            c = 0.5 * (ti * g + g)
        else:
            tf = jnp.tanh(gates[:, _D:2 * _D])
            c = 0.5 * (tf * c + c + ti * g + g)
        to = jnp.tanh(gates[:, 3 * _D:])
        tc = jnp.tanh(c)
        h2 = (to * tc + tc).astype(bf)
    cat = jnp.concatenate([h_ref[...], h2], axis=1)
    out_ref[...] = jax.nn.relu(
        jnp.dot(cat, wcomb_ref[...], preferred_element_type=jnp.float32)
        + bout_ref[...])


_tc_layer = pl.pallas_call(
    _lstm_body,
    grid=(_N // _BN,),
    in_specs=[
        pl.BlockSpec((_DEG, _BN, _D), lambda i: (0, i, 0)),
        pl.BlockSpec((_BN, _D), lambda i: (i, 0)),
        pl.BlockSpec((2 * _D, 4 * _D), lambda i: (0, 0)),
        pl.BlockSpec((1, 4 * _D), lambda i: (0, 0)),
        pl.BlockSpec((2 * _D, _D), lambda i: (0, 0)),
        pl.BlockSpec((1, _D), lambda i: (0, 0)),
    ],
    out_specs=pl.BlockSpec((_BN, _D), lambda i: (i, 0)),
    out_shape=jax.ShapeDtypeStruct((_N, _D), jnp.float32),
)


def kernel(x, edge_index, Wih, Whh, bih, bhh, Wself, Wneigh, b):
    src = edge_index[0]
    src_perm = jnp.pad(src.reshape(_N, _DEG).T, ((0, 0), (0, _NPAD - _N)))
    src_perm = src_perm.reshape(_NW, _CPW, _CHUNK)
    bf = jnp.bfloat16
    col = jnp.concatenate(
        [jnp.full((1, _D), 0.5), jnp.full((1, _D), 0.5),
         jnp.ones((1, _D)), jnp.full((1, _D), 0.5)], axis=1)
    row = jnp.concatenate(
        [jnp.ones((_D, 1)), jnp.full((_D, 1), 0.5)], axis=0)
    Wcat = (jnp.concatenate(
        [jnp.swapaxes(Wih, 1, 2), jnp.swapaxes(Whh, 1, 2)], axis=1)
        * col[None] * row[None]).astype(bf)
    Wcomb = (jnp.concatenate(
        [jnp.swapaxes(Wself, 1, 2), jnp.swapaxes(Wneigh, 1, 2)], axis=1)
        * row[None]).astype(bf)
    bias = (bih + bhh).reshape(_L, 1, 4 * _D) * col[None]
    bout = b.reshape(_L, 1, _D)

    h = x
    for l in range(_L):
        msg = _make_sc_gather()(h, src_perm).reshape(_DEG, _NPAD, _D)
        h = _tc_layer(msg, h.astype(bf), Wcat[l], bias[l], Wcomb[l], bout[l])
    return h


# speedup vs baseline: 2.0673x; 2.0673x over previous
import functools

import jax
import jax.numpy as jnp
from jax import lax
from jax.experimental import pallas as pl
from jax.experimental.pallas import tpu as pltpu
from jax.experimental.pallas import tpu_sc as plsc

_N = 10000
_DEG = 16
_D = 128
_L = 3

_NC, _NS = 2, 16
_NW = _NC * _NS
_CHUNK = 128
_NPAD = 10240
_CPW = (_NPAD * _DEG) // _NW // _CHUNK


@functools.cache
def _make_sc_gather():
    mesh = plsc.VectorSubcoreMesh(
        core_axis_name="c", subcore_axis_name="s",
        num_cores=_NC, num_subcores=_NS)

    @functools.partial(
        pl.kernel,
        mesh=mesh,
        out_type=jax.ShapeDtypeStruct((_NW * _CPW, _CHUNK, _D), jnp.float32),
        scratch_types=[
            pltpu.VMEM((_CPW, _CHUNK), jnp.int32),
            pltpu.VMEM((4, _CHUNK, _D), jnp.float32),
            pltpu.SemaphoreType.DMA,
            pltpu.SemaphoreType.DMA,
            pltpu.SemaphoreType.DMA,
            pltpu.SemaphoreType.DMA,
        ],
    )
    def _sc_gather(table_hbm, idx_hbm, out_hbm, idx_all, rows, s0, s1, s2, s3):
        wid = lax.axis_index("s") * _NC + lax.axis_index("c")
        base = wid * _CPW
        sem = (s0, s1, s2, s3)
        pltpu.sync_copy(idx_hbm.at[wid], idx_all)
        gd = [None] * 4
        wd = [None] * 4
        for j in range(2):
            gd[j] = pltpu.async_copy(table_hbm.at[idx_all.at[j]], rows.at[j],
                                     sem[j])
        for j in range(_CPW):
            b = j & 3
            gd[b].wait()
            nj = j + 2
            if nj < _CPW:
                nb = nj & 3
                if wd[nb] is not None:
                    wd[nb].wait()
                    wd[nb] = None
                gd[nb] = pltpu.async_copy(table_hbm.at[idx_all.at[nj]],
                                          rows.at[nb], sem[nb])
            wd[b] = pltpu.async_copy(rows.at[b], out_hbm.at[base + j], sem[b])
        for b in range(4):
            if wd[b] is not None:
                wd[b].wait()

    return _sc_gather


_BN = 1000


def _lstm_body(msg_ref, h_ref, wcat_ref, bias_ref, wcomb_ref, bout_ref,
               out_ref):
    bf = jnp.bfloat16
    c = None
    h2 = jnp.zeros((_BN, _D), bf)
    for t in range(_DEG):
        cat = jnp.concatenate([msg_ref[t].astype(bf), h2], axis=1)
        gates = jnp.dot(cat, wcat_ref[...],
                        preferred_element_type=jnp.float32) + bias_ref[...]
        ti = jnp.tanh(gates[:, :_D])
        g = jnp.tanh(gates[:, 2 * _D:3 * _D])
        if c is None:
            c = 0.5 * (ti * g + g)
        else:
            tf = jnp.tanh(gates[:, _D:2 * _D])
            c = 0.5 * (tf * c + c + ti * g + g)
        to = jnp.tanh(gates[:, 3 * _D:])
        tc = jnp.tanh(c)
        h2 = (to * tc + tc).astype(bf)
    cat = jnp.concatenate([h_ref[...], h2], axis=1)
    out_ref[...] = jax.nn.relu(
        jnp.dot(cat, wcomb_ref[...], preferred_element_type=jnp.float32)
        + bout_ref[...])


_tc_layer = pl.pallas_call(
    _lstm_body,
    grid=(_N // _BN,),
    in_specs=[
        pl.BlockSpec((_DEG, _BN, _D), lambda i: (0, i, 0)),
        pl.BlockSpec((_BN, _D), lambda i: (i, 0)),
        pl.BlockSpec((2 * _D, 4 * _D), lambda i: (0, 0)),
        pl.BlockSpec((1, 4 * _D), lambda i: (0, 0)),
        pl.BlockSpec((2 * _D, _D), lambda i: (0, 0)),
        pl.BlockSpec((1, _D), lambda i: (0, 0)),
    ],
    out_specs=pl.BlockSpec((_BN, _D), lambda i: (i, 0)),
    out_shape=jax.ShapeDtypeStruct((_N, _D), jnp.float32),
)


def kernel(x, edge_index, Wih, Whh, bih, bhh, Wself, Wneigh, b):
    src = edge_index[0]
    pad_idx = jnp.broadcast_to(jnp.arange(_NPAD - _N, dtype=src.dtype),
                               (_DEG, _NPAD - _N))
    src_perm = jnp.concatenate([src.reshape(_N, _DEG).T, pad_idx], axis=1)
    src_perm = src_perm.reshape(_NW, _CPW, _CHUNK)
    bf = jnp.bfloat16
    col = jnp.concatenate(
        [jnp.full((1, _D), 0.5), jnp.full((1, _D), 0.5),
         jnp.ones((1, _D)), jnp.full((1, _D), 0.5)], axis=1)
    row = jnp.concatenate(
        [jnp.ones((_D, 1)), jnp.full((_D, 1), 0.5)], axis=0)
    Wcat = (jnp.concatenate(
        [jnp.swapaxes(Wih, 1, 2), jnp.swapaxes(Whh, 1, 2)], axis=1)
        * col[None] * row[None]).astype(bf)
    Wcomb = (jnp.concatenate(
        [jnp.swapaxes(Wself, 1, 2), jnp.swapaxes(Wneigh, 1, 2)], axis=1)
        * row[None]).astype(bf)
    bias = (bih + bhh).reshape(_L, 1, 4 * _D) * col[None]
    bout = b.reshape(_L, 1, _D)

    h = x
    for l in range(_L):
        msg = _make_sc_gather()(h, src_perm).reshape(_DEG, _NPAD, _D)
        h = _tc_layer(msg, h.astype(bf), Wcat[l], bias[l], Wcomb[l], bout[l])
    return h


# speedup vs baseline: 2.1534x; 1.0417x over previous
import functools

import jax
import jax.numpy as jnp
from jax import lax
from jax.experimental import pallas as pl
from jax.experimental.pallas import tpu as pltpu
from jax.experimental.pallas import tpu_sc as plsc

_N = 10000
_DEG = 16
_D = 128
_L = 3

_NH = _N // 2
_NC, _NS = 2, 16
_NW = _NC * _NS
_CHUNK = 128
_NPADH = 5120
_CPW = (_NPADH * _DEG) // _NW // _CHUNK


@functools.cache
def _make_sc_gather():
    mesh = plsc.VectorSubcoreMesh(
        core_axis_name="c", subcore_axis_name="s",
        num_cores=_NC, num_subcores=_NS)

    @functools.partial(
        pl.kernel,
        mesh=mesh,
        out_type=jax.ShapeDtypeStruct((_NW * _CPW, _CHUNK, _D), jnp.float32),
        scratch_types=[
            pltpu.VMEM((_CPW, _CHUNK), jnp.int32),
            pltpu.VMEM((4, _CHUNK, _D), jnp.float32),
            pltpu.SemaphoreType.DMA,
            pltpu.SemaphoreType.DMA,
            pltpu.SemaphoreType.DMA,
            pltpu.SemaphoreType.DMA,
        ],
    )
    def _sc_gather(table_hbm, idx_hbm, out_hbm, idx_all, rows, s0, s1, s2, s3):
        wid = lax.axis_index("s") * _NC + lax.axis_index("c")
        base = wid * _CPW
        sem = (s0, s1, s2, s3)
        pltpu.sync_copy(idx_hbm.at[wid], idx_all)
        gd = [None] * 4
        wd = [None] * 4
        for j in range(2):
            gd[j] = pltpu.async_copy(table_hbm.at[idx_all.at[j]], rows.at[j],
                                     sem[j])
        for j in range(_CPW):
            b = j & 3
            gd[b].wait()
            nj = j + 2
            if nj < _CPW:
                nb = nj & 3
                if wd[nb] is not None:
                    wd[nb].wait()
                    wd[nb] = None
                gd[nb] = pltpu.async_copy(table_hbm.at[idx_all.at[nj]],
                                          rows.at[nb], sem[nb])
            wd[b] = pltpu.async_copy(rows.at[b], out_hbm.at[base + j], sem[b])
        for b in range(4):
            if wd[b] is not None:
                wd[b].wait()

    return _sc_gather


_BN = 1000
_GRIDH = _NH // _BN


def _lstm_body(msg_ref, h_ref, wcat_ref, bias_ref, wcomb_ref, bout_ref,
               out_ref):
    bf = jnp.bfloat16
    c = None
    h2 = jnp.zeros((_BN, _D), bf)
    for t in range(_DEG):
        cat = jnp.concatenate([msg_ref[t].astype(bf), h2], axis=1)
        gates = jnp.dot(cat, wcat_ref[...],
                        preferred_element_type=jnp.float32) + bias_ref[...]
        ti = jnp.tanh(gates[:, :_D])
        g = jnp.tanh(gates[:, 2 * _D:3 * _D])
        if c is None:
            c = 0.5 * (ti * g + g)
        else:
            tf = jnp.tanh(gates[:, _D:2 * _D])
            c = 0.5 * (tf * c + c + ti * g + g)
        to = jnp.tanh(gates[:, 3 * _D:])
        tc = jnp.tanh(c)
        h2 = (to * tc + tc).astype(bf)
    cat = jnp.concatenate([h_ref[...], h2], axis=1)
    out_ref[...] = jax.nn.relu(
        jnp.dot(cat, wcomb_ref[...], preferred_element_type=jnp.float32)
        + bout_ref[...])


def _make_tc_layer(half):
    off = half * _GRIDH
    return pl.pallas_call(
        _lstm_body,
        grid=(_GRIDH,),
        in_specs=[
            pl.BlockSpec((_DEG, _BN, _D), lambda i: (0, i, 0)),
            pl.BlockSpec((_BN, _D), lambda i: (i + off, 0)),
            pl.BlockSpec((2 * _D, 4 * _D), lambda i: (0, 0)),
            pl.BlockSpec((1, 4 * _D), lambda i: (0, 0)),
            pl.BlockSpec((2 * _D, _D), lambda i: (0, 0)),
            pl.BlockSpec((1, _D), lambda i: (0, 0)),
        ],
        out_specs=pl.BlockSpec((_BN, _D), lambda i: (i, 0)),
        out_shape=jax.ShapeDtypeStruct((_NH, _D), jnp.float32),
    )


_tc_half = (_make_tc_layer(0), _make_tc_layer(1))


def kernel(x, edge_index, Wih, Whh, bih, bhh, Wself, Wneigh, b):
    src = edge_index[0]
    srcs = src.reshape(_N, _DEG)
    pad_idx = jnp.broadcast_to(jnp.arange(_NPADH - _NH, dtype=src.dtype),
                               (_DEG, _NPADH - _NH))
    perms = []
    for half in range(2):
        s = srcs[half * _NH:(half + 1) * _NH].T
        perms.append(jnp.concatenate([s, pad_idx], axis=1)
                     .reshape(_NW, _CPW, _CHUNK))

    bf = jnp.bfloat16
    col = jnp.concatenate(
        [jnp.full((1, _D), 0.5), jnp.full((1, _D), 0.5),
         jnp.ones((1, _D)), jnp.full((1, _D), 0.5)], axis=1)
    row = jnp.concatenate(
        [jnp.ones((_D, 1)), jnp.full((_D, 1), 0.5)], axis=0)
    Wcat = (jnp.concatenate(
        [jnp.swapaxes(Wih, 1, 2), jnp.swapaxes(Whh, 1, 2)], axis=1)
        * col[None] * row[None]).astype(bf)
    Wcomb = (jnp.concatenate(
        [jnp.swapaxes(Wself, 1, 2), jnp.swapaxes(Wneigh, 1, 2)], axis=1)
        * row[None]).astype(bf)
    bias = (bih + bhh).reshape(_L, 1, 4 * _D) * col[None]
    bout = b.reshape(_L, 1, _D)

    h = x
    for l in range(_L):
        gather = _make_sc_gather()
        msgs = [gather(h, perms[half]).reshape(_DEG, _NPADH, _D)
                for half in range(2)]
        h_bf = h.astype(bf)
        h = jnp.concatenate(
            [_tc_half[half](msgs[half], h_bf, Wcat[l], bias[l], Wcomb[l],
                            bout[l])
             for half in range(2)], axis=0)
    return h


# speedup vs baseline: 2.1823x; 1.0134x over previous
import functools

import jax
import jax.numpy as jnp
from jax import lax
from jax.experimental import pallas as pl
from jax.experimental.pallas import tpu as pltpu
from jax.experimental.pallas import tpu_sc as plsc

_N = 10000
_DEG = 16
_D = 128
_L = 3

_HALVES = 2
_NH = _N // _HALVES
_NC, _NS = 2, 16
_NW = _NC * _NS
_CHUNK = 128
_NPADH = _NH + 120
_CPW = (_NPADH * _DEG) // _NW // _CHUNK


@functools.cache
def _make_sc_gather():
    mesh = plsc.VectorSubcoreMesh(
        core_axis_name="c", subcore_axis_name="s",
        num_cores=_NC, num_subcores=_NS)

    @functools.partial(
        pl.kernel,
        mesh=mesh,
        out_type=jax.ShapeDtypeStruct((_NW * _CPW, _CHUNK, _D), jnp.float32),
        scratch_types=[
            pltpu.VMEM((_CPW, _CHUNK), jnp.int32),
            pltpu.VMEM((6, _CHUNK, _D), jnp.float32),
            pltpu.SemaphoreType.DMA,
            pltpu.SemaphoreType.DMA,
            pltpu.SemaphoreType.DMA,
            pltpu.SemaphoreType.DMA,
            pltpu.SemaphoreType.DMA,
            pltpu.SemaphoreType.DMA,
        ],
    )
    def _sc_gather(table_hbm, idx_hbm, out_hbm, idx_all, rows,
                   s0, s1, s2, s3, s4, s5):
        wid = lax.axis_index("s") * _NC + lax.axis_index("c")
        base = wid * _CPW
        sem = (s0, s1, s2, s3, s4, s5)
        pltpu.sync_copy(idx_hbm.at[wid], idx_all)
        gd = [None] * 6
        wd = [None] * 6
        for j in range(3):
            gd[j] = pltpu.async_copy(table_hbm.at[idx_all.at[j]], rows.at[j],
                                     sem[j])
        for j in range(_CPW):
            b = j % 6
            gd[b].wait()
            nj = j + 3
            if nj < _CPW:
                nb = nj % 6
                if wd[nb] is not None:
                    wd[nb].wait()
                    wd[nb] = None
                gd[nb] = pltpu.async_copy(table_hbm.at[idx_all.at[nj]],
                                          rows.at[nb], sem[nb])
            wd[b] = pltpu.async_copy(rows.at[b], out_hbm.at[base + j], sem[b])
        for b in range(6):
            if wd[b] is not None:
                wd[b].wait()

    return _sc_gather


_BN = 1000
_GRIDH = _NH // _BN


def _lstm_body(msg_ref, h_ref, wcat_ref, bias_ref, wcomb_ref, bout_ref,
               out_ref):
    bf = jnp.bfloat16
    c = None
    h2 = jnp.zeros((_BN, _D), bf)
    for t in range(_DEG):
        cat = jnp.concatenate([msg_ref[t].astype(bf), h2], axis=1)
        gates = jnp.dot(cat, wcat_ref[...],
                        preferred_element_type=jnp.float32) + bias_ref[...]
        ti = jnp.tanh(gates[:, :_D])
        g = jnp.tanh(gates[:, 2 * _D:3 * _D])
        if c is None:
            c = 0.5 * (ti * g + g)
        else:
            tf = jnp.tanh(gates[:, _D:2 * _D])
            c = 0.5 * (tf * c + c + ti * g + g)
        to = jnp.tanh(gates[:, 3 * _D:])
        tc = jnp.tanh(c)
        h2 = (to * tc + tc).astype(bf)
    cat = jnp.concatenate([h_ref[...].astype(bf), h2], axis=1)
    out_ref[...] = jax.nn.relu(
        jnp.dot(cat, wcomb_ref[...], preferred_element_type=jnp.float32)
        + bout_ref[...])


def _make_tc_layer(half):
    off = half * _GRIDH
    return pl.pallas_call(
        _lstm_body,
        grid=(_GRIDH,),
        in_specs=[
            pl.BlockSpec((_DEG, _BN, _D), lambda i: (0, i, 0)),
            pl.BlockSpec((_BN, _D), lambda i: (i + off, 0)),
            pl.BlockSpec((2 * _D, 4 * _D), lambda i: (0, 0)),
            pl.BlockSpec((1, 4 * _D), lambda i: (0, 0)),
            pl.BlockSpec((2 * _D, _D), lambda i: (0, 0)),
            pl.BlockSpec((1, _D), lambda i: (0, 0)),
        ],
        out_specs=pl.BlockSpec((_BN, _D), lambda i: (i, 0)),
        out_shape=jax.ShapeDtypeStruct((_NH, _D), jnp.float32),
    )


_tc_half = tuple(_make_tc_layer(i) for i in range(_HALVES))


def kernel(x, edge_index, Wih, Whh, bih, bhh, Wself, Wneigh, b):
    src = edge_index[0]
    srcs = src.reshape(_N, _DEG)
    pad_idx = jnp.broadcast_to(jnp.arange(_NPADH - _NH, dtype=src.dtype),
                               (_DEG, _NPADH - _NH))
    perms = []
    for half in range(_HALVES):
        s = srcs[half * _NH:(half + 1) * _NH].T
        perms.append(jnp.concatenate([s, pad_idx], axis=1)
                     .reshape(_NW, _CPW, _CHUNK))

    bf = jnp.bfloat16
    col = jnp.concatenate(
        [jnp.full((1, _D), 0.5), jnp.full((1, _D), 0.5),
         jnp.ones((1, _D)), jnp.full((1, _D), 0.5)], axis=1)
    row = jnp.concatenate(
        [jnp.ones((_D, 1)), jnp.full((_D, 1), 0.5)], axis=0)
    Wcat = (jnp.concatenate(
        [jnp.swapaxes(Wih, 1, 2), jnp.swapaxes(Whh, 1, 2)], axis=1)
        * col[None] * row[None]).astype(bf)
    Wcomb = (jnp.concatenate(
        [jnp.swapaxes(Wself, 1, 2), jnp.swapaxes(Wneigh, 1, 2)], axis=1)
        * row[None]).astype(bf)
    bias = (bih + bhh).reshape(_L, 1, 4 * _D) * col[None]
    bout = b.reshape(_L, 1, _D)

    h = x
    for l in range(_L):
        gather = _make_sc_gather()
        msgs = [gather(h, perms[half]).reshape(_DEG, _NPADH, _D)
                for half in range(_HALVES)]
        h = jnp.concatenate(
            [_tc_half[half](msgs[half], h, Wcat[l], bias[l], Wcomb[l],
                            bout[l])
             for half in range(_HALVES)], axis=0)
    return h
